# baseline (speedup 1.0000x reference)
import functools
import math

import jax
import jax.numpy as jnp
from jax import lax
from jax.experimental import pallas as pl
from jax.experimental.pallas import tpu as pltpu
from jax.experimental.pallas import tpu_sc as plsc

NUM_EMBEDDINGS = 1000000
D_MODEL = 64
SCALE = math.sqrt(D_MODEL)

B_ROWS = 4096
B_COLS = 50
N_TOTAL = B_ROWS * B_COLS

_INFO = plsc.get_sparse_core_info()
NC = _INFO.num_cores
NS = _INFO.num_subcores
NW = NC * NS
LANES = _INFO.num_lanes

SUB = 128
K_PER_CHUNK = 10
CHUNK = SUB * K_PER_CHUNK
PER_W = N_TOTAL // NW
CHUNKS_PER_W = PER_W // CHUNK
IDX_ROWS = N_TOTAL // SUB
ROWS_PER_W = PER_W // SUB


def _body(table_hbm, idx_hbm, out_hbm, idx_v, rows_v, sem):
    wid = lax.axis_index("s") * NC + lax.axis_index("c")
    out_base = wid * PER_W

    pltpu.sync_copy(idx_hbm.at[wid], idx_v)

    def do_chunk(i, _):
        for j in range(K_PER_CHUNK):
            pltpu.async_copy(table_hbm.at[idx_v.at[i * K_PER_CHUNK + j]],
                             rows_v.at[pl.ds(j * SUB, SUB)], sem)
        for j in range(K_PER_CHUNK):
            pltpu.make_async_copy(table_hbm.at[idx_v.at[i * K_PER_CHUNK + j]],
                                  rows_v.at[pl.ds(j * SUB, SUB)], sem).wait()
        def scale_row(r, _):
            for v in range(D_MODEL // LANES):
                sl = pl.ds(v * LANES, LANES)
                rows_v[r, sl] = rows_v[r, sl] * SCALE
            return 0
        lax.fori_loop(0, CHUNK, scale_row, 0)
        pltpu.sync_copy(rows_v,
                        out_hbm.at[pl.ds(out_base + i * CHUNK, CHUNK)])
        return 0

    lax.fori_loop(0, CHUNKS_PER_W, do_chunk, 0)


@jax.jit
def _embed(table, idx2d):
    mesh = plsc.VectorSubcoreMesh(core_axis_name="c", subcore_axis_name="s")
    kern = pl.kernel(
        _body,
        out_type=jax.ShapeDtypeStruct((N_TOTAL, D_MODEL), jnp.float32),
        mesh=mesh,
        scratch_types=[
            pltpu.VMEM((ROWS_PER_W, SUB), jnp.int32),
            pltpu.VMEM((CHUNK, D_MODEL), jnp.float32),
            pltpu.SemaphoreType.DMA,
        ],
        compiler_params=pltpu.CompilerParams(use_tc_tiling_on_sc=False),
    )
    return kern(table, idx2d)


def kernel(inputs, table):
    idx3d = inputs.reshape(NW, ROWS_PER_W, SUB).astype(jnp.int32)
    out = _embed(table, idx3d)
    return out.reshape(B_ROWS, B_COLS, D_MODEL)
